# baseline (speedup 1.0000x reference)
import functools

import jax
import jax.numpy as jnp
from jax import lax
from jax.experimental import pallas as pl
from jax.experimental.pallas import tpu as pltpu
from jax.experimental.pallas import tpu_sc as plsc

_LANES = 16
_IDX_COLS = 128


def _make_kernel(B, D, NC, NS):
    NW = NC * NS
    BW = B // NW
    NIDX = BW // _IDX_COLS
    NGRP = BW // _LANES
    mesh = plsc.VectorSubcoreMesh(core_axis_name="c", subcore_axis_name="s")

    @functools.partial(
        pl.kernel,
        mesh=mesh,
        out_type=jax.ShapeDtypeStruct((B,), jnp.float32),
        compiler_params=pltpu.CompilerParams(
            needs_layout_passes=False, use_tc_tiling_on_sc=False),
        scratch_types=[
            pltpu.VMEM((NIDX, _IDX_COLS), jnp.int32),
            pltpu.VMEM((NIDX, _IDX_COLS), jnp.int32),
            pltpu.VMEM((BW, D), jnp.float32),
            pltpu.VMEM((BW, D), jnp.float32),
            pltpu.VMEM((BW,), jnp.float32),
            pltpu.SemaphoreType.DMA,
        ],
    )
    def k(user_hbm, item_hbm, uemb_hbm, iemb_hbm, out_hbm,
          uidx, iidx, urows, irows, outv, sem):
        wid = lax.axis_index("s") * NC + lax.axis_index("c")
        pltpu.sync_copy(user_hbm.at[pl.ds(wid * NIDX, NIDX)], uidx)
        pltpu.sync_copy(item_hbm.at[pl.ds(wid * NIDX, NIDX)], iidx)
        copies = []
        for c in range(NIDX):
            dst = pl.ds(c * _IDX_COLS, _IDX_COLS)
            copies.append(pltpu.async_copy(
                uemb_hbm.at[uidx.at[c]], urows.at[dst], sem))
            copies.append(pltpu.async_copy(
                iemb_hbm.at[iidx.at[c]], irows.at[dst], sem))
        for cp in copies:
            cp.wait()

        lane = lax.iota(jnp.int32, _LANES)

        def body(g, carry):
            acc = jnp.zeros((_LANES,), jnp.float32)
            for k in range(_LANES):
                r = g * _LANES + k
                p = jnp.zeros((_LANES,), jnp.float32)
                for c in range(D // _LANES):
                    sl = pl.ds(c * _LANES, _LANES)
                    p = p + urows[r, sl] * irows[r, sl]
                acc = jnp.where(lane == k, jnp.sum(p), acc)
            outv[pl.ds(g * _LANES, _LANES)] = acc
            return carry

        lax.fori_loop(0, NGRP, body, 0)
        pltpu.sync_copy(outv, out_hbm.at[pl.ds(wid * BW, BW)])

    return k


@jax.jit
def kernel(user, item, user_emb, item_emb):
    B = user.shape[0]
    D = user_emb.shape[1]
    info = plsc.get_sparse_core_info()
    k = _make_kernel(B, D, info.num_cores, info.num_subcores)
    user2d = user.astype(jnp.int32).reshape(B // _IDX_COLS, _IDX_COLS)
    item2d = item.astype(jnp.int32).reshape(B // _IDX_COLS, _IDX_COLS)
    return k(user2d, item2d, user_emb, item_emb)
